# baseline (speedup 1.0000x reference)
import jax
import jax.numpy as jnp
from jax.experimental import pallas as pl


def _gcn_conv(x, src, dst, edge_weight, W, b):
    n = x.shape[0]
    loop = jnp.arange(n)
    s = jnp.concatenate([src, loop])
    d = jnp.concatenate([dst, loop])
    w = jnp.concatenate([edge_weight, jnp.ones((n,), dtype=x.dtype)])
    deg = jnp.zeros((n,), dtype=x.dtype).at[d].add(w)
    dis = jnp.where(deg > 0, 1.0 / jnp.sqrt(deg), 0.0)
    norm = dis[s] * w * dis[d]
    h = x @ W.T
    out = jnp.zeros_like(h).at[d].add(h[s] * norm[:, None])
    return out + b


def _layer_norm(x, g, b):
    mu = jnp.mean(x, axis=-1, keepdims=True)
    var = jnp.mean((x - mu) ** 2, axis=-1, keepdims=True)
    return (x - mu) / jnp.sqrt(var + 1e-5) * g + b


def _gelu_exact(x):
    return 0.5 * x * (1.0 + jax.lax.erf(x * 0.7071067811865476))


def _pool_kernel(hsum_ref, wp0_ref, bp0_ref, wp1_ref, bp1_ref, out_ref):
    ge = hsum_ref[...] / 10000.0
    t = ge @ wp0_ref[...].T + bp0_ref[...]
    t = _gelu_exact(t)
    out_ref[...] = t @ wp1_ref[...].T + bp1_ref[...]


def kernel(x, edge_index, edge_attr, W_gcn0, b_gcn0, W_gcn1, b_gcn1, W_gcn2,
           b_gcn2, W_res0, b_res0, W_res2, b_res2, ln_g0, ln_b0, ln_g1, ln_b1,
           ln_g2, ln_b2, W_p0, b_p0, W_p1, b_p1):
    src, dst = edge_index[0], edge_index[1]
    ew = edge_attr[:, 0]
    h = x
    r = h @ W_res0.T + b_res0
    h = _gcn_conv(h, src, dst, ew, W_gcn0, b_gcn0) + r
    h = _layer_norm(h, ln_g0, ln_b0)
    h = jax.nn.gelu(h, approximate=False)
    r = h
    h = _gcn_conv(h, src, dst, ew, W_gcn1, b_gcn1) + r
    h = _layer_norm(h, ln_g1, ln_b1)
    h = jax.nn.gelu(h, approximate=False)
    r = h @ W_res2.T + b_res2
    h = _gcn_conv(h, src, dst, ew, W_gcn2, b_gcn2) + r
    h = _layer_norm(h, ln_g2, ln_b2)

    hsum = jnp.sum(h, axis=0, keepdims=True)
    ge = pl.pallas_call(
        _pool_kernel,
        out_shape=jax.ShapeDtypeStruct((1, 256), jnp.float32),
    )(hsum, W_p0, b_p0.reshape(1, 256), W_p1, b_p1.reshape(1, 256))
    return h, ge


# speedup vs baseline: 3.7330x; 3.7330x over previous
import functools

import jax
import jax.numpy as jnp
from jax import lax
from jax.experimental import pallas as pl
from jax.experimental.pallas import tpu as pltpu
from jax.experimental.pallas import tpu_sc as plsc

N = 10000
NP = 10112
D = 256
H = 512
E = 160000
NC = 2
NS = 16
NW = NC * NS
EPT = 5504
CK = EPT // 128
EPAD = NW * EPT
RPT = NP // NS

_mesh = plsc.VectorSubcoreMesh(core_axis_name="c", subcore_axis_name="s")


def _gelu_exact(x):
    return 0.5 * x * (1.0 + lax.erf(x * 0.7071067811865476))


@functools.partial(
    pl.kernel,
    out_type=jax.ShapeDtypeStruct((NC, NP, 128), jnp.float32),
    mesh=_mesh,
    compiler_params=pltpu.CompilerParams(needs_layout_passes=False),
    scratch_types=[
        pltpu.VMEM((CK, 128), jnp.int32),
        pltpu.VMEM((CK, 128), jnp.float32),
        pltpu.VMEM((128, 128), jnp.float32),
        pltpu.VMEM((128, 128), jnp.float32),
        pltpu.VMEM_SHARED((NP, 128), jnp.float32),
    ],
)
def _deg_kernel(gdst3, gw, out, dst2d, w2d, wrows, zbuf, acc):
    cid = lax.axis_index("c")
    sid = lax.axis_index("s")
    wid = sid * NC + cid
    pltpu.sync_copy(gdst3.at[wid], dst2d)
    pltpu.sync_copy(gw.at[wid], w2d)
    zero16 = jnp.zeros((16,), jnp.float32)

    def _zw(i, _):
        def _zj(j, _):
            wrows[i, pl.ds(j * 16, 16)] = zero16
            zbuf[i, pl.ds(j * 16, 16)] = zero16
            return 0
        lax.fori_loop(0, 8, _zj, 0)
        return 0
    lax.fori_loop(0, 128, _zw, 0)
    for j in range(4):
        pltpu.sync_copy(zbuf, acc.at[pl.ds(sid * RPT + j * 128, 128)])
    pltpu.sync_copy(zbuf.at[pl.ds(0, 120)],
                    acc.at[pl.ds(sid * RPT + 512, 120)])
    plsc.subcore_barrier()

    lane = lax.iota(jnp.int32, 16)
    col0 = jnp.zeros((16,), jnp.int32)

    def _chunk(k, _):
        def _grp(g, _):
            w16 = w2d[k, pl.ds(g * 16, 16)]
            plsc.store_scatter(wrows, [g * 16 + lane, col0], w16)
            return 0
        lax.fori_loop(0, 8, _grp, 0)
        pltpu.sync_copy(wrows, acc.at[dst2d.at[k]], add=True)
        return 0
    lax.fori_loop(0, CK, _chunk, 0)
    plsc.subcore_barrier()
    pltpu.sync_copy(acc.at[pl.ds(sid * RPT, RPT)],
                    out.at[cid, pl.ds(sid * RPT, RPT)])


@functools.partial(
    pl.kernel,
    out_type=jax.ShapeDtypeStruct((NW, CK, 128), jnp.float32),
    mesh=_mesh,
    compiler_params=pltpu.CompilerParams(needs_layout_passes=False),
    scratch_types=[
        pltpu.VMEM((NP // 128, 128), jnp.float32),
        pltpu.VMEM((CK, 128), jnp.int32),
        pltpu.VMEM((CK, 128), jnp.int32),
        pltpu.VMEM((CK, 128), jnp.float32),
        pltpu.VMEM((CK, 128), jnp.float32),
    ],
)
def _norm_kernel(dis_hbm, gsrc, gdst, gw, out, dis2d, src2d, dst2d, w2d, nbuf):
    cid = lax.axis_index("c")
    sid = lax.axis_index("s")
    wid = sid * NC + cid
    pltpu.sync_copy(dis_hbm, dis2d)
    pltpu.sync_copy(gsrc.at[wid], src2d)
    pltpu.sync_copy(gdst.at[wid], dst2d)
    pltpu.sync_copy(gw.at[wid], w2d)

    def _chunk(k, _):
        def _grp(g, _):
            s16 = src2d[k, pl.ds(g * 16, 16)]
            d16 = dst2d[k, pl.ds(g * 16, 16)]
            w16 = w2d[k, pl.ds(g * 16, 16)]
            ds_ = plsc.load_gather(dis2d, [lax.div(s16, 128), lax.rem(s16, 128)])
            dd_ = plsc.load_gather(dis2d, [lax.div(d16, 128), lax.rem(d16, 128)])
            nbuf[k, pl.ds(g * 16, 16)] = ds_ * w16 * dd_
            return 0
        lax.fori_loop(0, 8, _grp, 0)
        return 0
    lax.fori_loop(0, CK, _chunk, 0)
    pltpu.sync_copy(nbuf, out.at[wid])


def _make_spmm(nch):
    @functools.partial(
        pl.kernel,
        out_type=jax.ShapeDtypeStruct((NC, NP, nch * 128), jnp.float32),
        mesh=_mesh,
        compiler_params=pltpu.CompilerParams(needs_layout_passes=False),
        scratch_types=[
            pltpu.VMEM((CK, 128), jnp.int32),
            pltpu.VMEM((CK, 128), jnp.int32),
            pltpu.VMEM((CK, 128), jnp.float32),
            pltpu.VMEM((128, 128), jnp.float32),
            pltpu.VMEM_SHARED((NP, 128), jnp.float32),
            pltpu.SemaphoreType.DMA,
            pltpu.SemaphoreType.DMA,
        ],
    )
    def _spmm(xflat, gsrc, gdst3, gnorm, out,
              gidx2d, dst2d, norm2d, rows0, acc, sem0, sem1):
        cid = lax.axis_index("c")
        sid = lax.axis_index("s")
        wid = sid * NC + cid
        pltpu.sync_copy(gsrc.at[wid], gidx2d)
        pltpu.sync_copy(gdst3.at[wid], dst2d)
        pltpu.sync_copy(gnorm.at[wid], norm2d)
        zero16 = jnp.zeros((16,), jnp.float32)
        lane = lax.iota(jnp.int32, 16)

        def _start_gather(k, rows, sem):
            pltpu.async_copy(xflat.at[gidx2d.at[k, pl.ds(0, 64)]],
                             rows.at[pl.ds(0, 64)], sem)
            pltpu.async_copy(xflat.at[gidx2d.at[k, pl.ds(64, 64)]],
                             rows.at[pl.ds(64, 64)], sem1)

        def _wait_gather(rows, sem):
            pltpu.make_async_copy(xflat.at[gidx2d.at[0, pl.ds(0, 64)]],
                                  rows.at[pl.ds(0, 64)], sem).wait()
            pltpu.make_async_copy(xflat.at[gidx2d.at[0, pl.ds(64, 64)]],
                                  rows.at[pl.ds(64, 64)], sem1).wait()

        def _scale_half(k, rows, base):
            def _grp(g0, _):
                g = base // 16 + g0
                n16 = norm2d[k, pl.ds(g * 16, 16)]

                for ee in range(16):
                    e = g * 16 + ee
                    nv = jnp.broadcast_to(n16[ee], (16,))
                    for j in range(8):
                        rows[e, pl.ds(j * 16, 16)] = (
                            rows[e, pl.ds(j * 16, 16)] * nv)
                return 0
            lax.fori_loop(0, 4, _grp, 0)

        for c in range(nch):
            def _gi(k, _):
                def _gg(g, _):
                    s16 = gidx2d[k, pl.ds(g * 16, 16)]
                    if c == 0:
                        gidx2d[k, pl.ds(g * 16, 16)] = s16 * nch
                    else:
                        gidx2d[k, pl.ds(g * 16, 16)] = s16 + 1
                    return 0
                lax.fori_loop(0, 8, _gg, 0)
                return 0
            lax.fori_loop(0, CK, _gi, 0)

            def _zz(i, _):
                def _zj(j, _):
                    rows0[i, pl.ds(j * 16, 16)] = zero16
                    return 0
                lax.fori_loop(0, 8, _zj, 0)
                return 0
            lax.fori_loop(0, 128, _zz, 0)
            for j in range(4):
                pltpu.sync_copy(rows0, acc.at[pl.ds(sid * RPT + j * 128, 128)])
            pltpu.sync_copy(rows0.at[pl.ds(0, 120)],
                            acc.at[pl.ds(sid * RPT + 512, 120)])
            plsc.subcore_barrier()

            _start_gather(0, rows0, sem0)

            def _chunk(k, _):
                pltpu.make_async_copy(
                    xflat.at[gidx2d.at[0, pl.ds(0, 64)]],
                    rows0.at[pl.ds(0, 64)], sem0).wait()
                _scale_half(k, rows0, 0)
                pltpu.sync_copy(rows0.at[pl.ds(0, 64)],
                                acc.at[dst2d.at[k, pl.ds(0, 64)]], add=True)

                @pl.when(k + 1 < CK)
                def _():
                    pltpu.async_copy(
                        xflat.at[gidx2d.at[k + 1, pl.ds(0, 64)]],
                        rows0.at[pl.ds(0, 64)], sem0)
                pltpu.make_async_copy(
                    xflat.at[gidx2d.at[0, pl.ds(64, 64)]],
                    rows0.at[pl.ds(64, 64)], sem1).wait()
                _scale_half(k, rows0, 64)
                pltpu.sync_copy(rows0.at[pl.ds(64, 64)],
                                acc.at[dst2d.at[k, pl.ds(64, 64)]], add=True)

                @pl.when(k + 1 < CK)
                def _():
                    pltpu.async_copy(
                        xflat.at[gidx2d.at[k + 1, pl.ds(64, 64)]],
                        rows0.at[pl.ds(64, 64)], sem1)
                return 0
            lax.fori_loop(0, CK, _chunk, 0)
            plsc.subcore_barrier()
            pltpu.sync_copy(acc.at[pl.ds(sid * RPT, RPT)],
                            out.at[cid, pl.ds(sid * RPT, RPT),
                                   pl.ds(c * 128, 128)])
            plsc.subcore_barrier()
    return _spmm


_spmm2 = _make_spmm(2)
_spmm4 = _make_spmm(4)


_CC = (((1,), (1,)), ((), ()))


def _dis_body(degp_ref, out_ref):
    deg = jnp.sum(degp_ref[...], axis=(0, 2))
    out_ref[...] = lax.rsqrt(jnp.maximum(deg, 1e-12)).reshape(NP // 128, 128)


def _layer_norm(t, g, b):
    mu = jnp.mean(t, axis=-1, keepdims=True)
    var = jnp.mean((t - mu) ** 2, axis=-1, keepdims=True)
    return (t - mu) * lax.rsqrt(var + 1e-5) * g + b


def _l0_body(p_ref, x_ref, w0_ref, wr0_ref, b_ref, g_ref, bt_ref, o_ref):
    agg = p_ref[0] + p_ref[1]
    t = lax.dot_general(agg, w0_ref[...], _CC, preferred_element_type=jnp.float32)
    t = t + lax.dot_general(x_ref[...], wr0_ref[...], _CC,
                            preferred_element_type=jnp.float32)
    t = t + b_ref[...]
    o_ref[...] = _gelu_exact(_layer_norm(t, g_ref[...], bt_ref[...]))


def _l1_body(p_ref, h1_ref, w1_ref, b1_ref, g_ref, bt_ref,
             w2_ref, wr2_ref, br2_ref, t2_ref, r2_ref):
    agg = p_ref[0] + p_ref[1]
    t = lax.dot_general(agg, w1_ref[...], _CC, preferred_element_type=jnp.float32)
    t = t + b1_ref[...] + h1_ref[...]
    h2 = _gelu_exact(_layer_norm(t, g_ref[...], bt_ref[...]))
    t2_ref[...] = lax.dot_general(h2, w2_ref[...], _CC,
                                  preferred_element_type=jnp.float32)
    r2_ref[...] = lax.dot_general(h2, wr2_ref[...], _CC,
                                  preferred_element_type=jnp.float32) + br2_ref[...]


def _l2_body(p_ref, r2_ref, b2_ref, g_ref, bt_ref, h3_ref, cs_ref):
    t = p_ref[0] + p_ref[1] + b2_ref[...] + r2_ref[...]
    h3 = _layer_norm(t, g_ref[...], bt_ref[...])
    h3_ref[...] = h3

    @pl.when(pl.program_id(0) == 0)
    def _():
        cs_ref[...] = jnp.zeros_like(cs_ref)
    cs_ref[...] += jnp.sum(h3, axis=0, keepdims=True)


def _pool_body(cs_ref, wp0_ref, bp0_ref, wp1_ref, bp1_ref, out_ref):
    ge = cs_ref[...] * (1.0 / N)
    t = _gelu_exact(lax.dot_general(ge, wp0_ref[...], _CC,
                                    preferred_element_type=jnp.float32)
                    + bp0_ref[...])
    out_ref[...] = lax.dot_general(t, wp1_ref[...], _CC,
                                   preferred_element_type=jnp.float32) + bp1_ref[...]


_BLK = 1000
_GRID = N // _BLK


def _full(shape):
    return pl.BlockSpec(shape, lambda i: tuple(0 for _ in shape))


def _l0_call(p, x, W0, Wr0, b, g, bt):
    return pl.pallas_call(
        _l0_body,
        grid=(_GRID,),
        in_specs=[
            pl.BlockSpec((NC, _BLK, D), lambda i: (0, i, 0)),
            pl.BlockSpec((_BLK, D), lambda i: (i, 0)),
            _full((H, D)), _full((H, D)), _full((1, H)), _full((1, H)),
            _full((1, H)),
        ],
        out_specs=pl.BlockSpec((_BLK, H), lambda i: (i, 0)),
        out_shape=jax.ShapeDtypeStruct((N, H), jnp.float32),
    )(p, x, W0, Wr0, b, g, bt)


def _l1_call(p, h1, W1, b1, g, bt, W2, Wr2, br2):
    return pl.pallas_call(
        _l1_body,
        grid=(_GRID,),
        in_specs=[
            pl.BlockSpec((NC, _BLK, H), lambda i: (0, i, 0)),
            pl.BlockSpec((_BLK, H), lambda i: (i, 0)),
            _full((H, H)), _full((1, H)), _full((1, H)), _full((1, H)),
            _full((D, H)), _full((D, H)), _full((1, D)),
        ],
        out_specs=[
            pl.BlockSpec((_BLK, D), lambda i: (i, 0)),
            pl.BlockSpec((_BLK, D), lambda i: (i, 0)),
        ],
        out_shape=[
            jax.ShapeDtypeStruct((N, D), jnp.float32),
            jax.ShapeDtypeStruct((N, D), jnp.float32),
        ],
    )(p, h1, W1, b1, g, bt, W2, Wr2, br2)


def _l2_call(p, r2, b2, g, bt):
    return pl.pallas_call(
        _l2_body,
        grid=(_GRID,),
        in_specs=[
            pl.BlockSpec((NC, _BLK, D), lambda i: (0, i, 0)),
            pl.BlockSpec((_BLK, D), lambda i: (i, 0)),
            _full((1, D)), _full((1, D)), _full((1, D)),
        ],
        out_specs=[
            pl.BlockSpec((_BLK, D), lambda i: (i, 0)),
            pl.BlockSpec((1, D), lambda i: (0, 0)),
        ],
        out_shape=[
            jax.ShapeDtypeStruct((N, D), jnp.float32),
            jax.ShapeDtypeStruct((1, D), jnp.float32),
        ],
    )(p, r2, b2, g, bt)


def kernel(x, edge_index, edge_attr, W_gcn0, b_gcn0, W_gcn1, b_gcn1, W_gcn2,
           b_gcn2, W_res0, b_res0, W_res2, b_res2, ln_g0, ln_b0, ln_g1, ln_b1,
           ln_g2, ln_b2, W_p0, b_p0, W_p1, b_p1):
    src = edge_index[0]
    dst = edge_index[1]
    ew = edge_attr[:, 0]
    loop = jnp.arange(N, dtype=jnp.int32)
    padi = jnp.zeros((EPAD - E - N,), jnp.int32)
    padf = jnp.zeros((EPAD - E - N,), jnp.float32)
    s_all = jnp.concatenate([src, loop, padi])
    d_all = jnp.concatenate([dst, loop, padi])
    w_all = jnp.concatenate([ew, jnp.ones((N,), jnp.float32), padf])
    gsrc = s_all.reshape(NW, CK, 128)
    gdst3 = d_all.reshape(NW, CK, 128)
    gw = w_all.reshape(NW, CK, 128)

    degp = _deg_kernel(gdst3, gw)
    dis = pl.pallas_call(
        _dis_body,
        out_shape=jax.ShapeDtypeStruct((NP // 128, 128), jnp.float32),
    )(degp)
    gnorm = _norm_kernel(dis, gsrc, gdst3, gw)

    p0 = _spmm2(x.reshape(N * 2, 128), gsrc, gdst3, gnorm)
    h1 = _l0_call(p0, x, W_gcn0, W_res0,
                  (b_gcn0 + b_res0).reshape(1, H),
                  ln_g0.reshape(1, H), ln_b0.reshape(1, H))
    p1 = _spmm4(h1.reshape(N * 4, 128), gsrc, gdst3, gnorm)
    t2, r2 = _l1_call(p1, h1, W_gcn1, b_gcn1.reshape(1, H),
                      ln_g1.reshape(1, H), ln_b1.reshape(1, H),
                      W_gcn2, W_res2, b_res2.reshape(1, D))
    p2 = _spmm2(t2.reshape(N * 2, 128), gsrc, gdst3, gnorm)
    h3, csum = _l2_call(p2, r2, b_gcn2.reshape(1, D),
                        ln_g2.reshape(1, D), ln_b2.reshape(1, D))
    ge = pl.pallas_call(
        _pool_body,
        out_shape=jax.ShapeDtypeStruct((1, D), jnp.float32),
    )(csum, W_p0, b_p0.reshape(1, D), W_p1, b_p1.reshape(1, D))
    return h3, ge
